# Initial kernel scaffold; baseline (speedup 1.0000x reference)
import jax
import jax.numpy as jnp
from jax.experimental import pallas as pl


def kernel(x, edge_index, edge_weight, W_e, u_e, c_e, b_e, W_lat, b_lat, W_dec, b_dec, W_d, u_d, c_d, b_d):
    raise NotImplementedError("write your pallas kernel here")


# speedup vs baseline: 12.3875x; 12.3875x over previous
import functools

import jax
import jax.numpy as jnp
import numpy as np
from jax import lax
from jax.experimental import pallas as pl
from jax.experimental.pallas import tpu as pltpu
from jax.experimental.pallas import tpu_sc as plsc

V = 10000
E = 160000
B = 4
M = 9
F_IN = 3
F_MID = 16
LATENT = 64

NC = 2
NS = 16
L = 16

CHUNK = 128
N_CHUNKS = E // CHUNK
STRIPE = 1280
YROWS = NS * STRIPE
PSI_W = 32
NEG = -1e30


def _lane_mask(shape, n, dim):
    return lax.broadcasted_iota(jnp.int32, shape, dim) < n


def _prep_enc_body(x_ref, u_ref, c_ref, g_ref, h_ref, hx_ref, psi_ref):
    xb = x_ref[0]
    a = jnp.dot(xb, u_ref[...], preferred_element_type=jnp.float32)
    vb = a.shape[0]
    msk = _lane_mask((vb, 16), M, 1)
    amax = jnp.max(jnp.where(msk, a, NEG), axis=1, keepdims=True)
    g = jnp.where(msk, jnp.exp(a - amax), 0.0)
    ca = c_ref[...][None, :] - a
    dmax = jnp.max(jnp.where(msk, ca, NEG), axis=1, keepdims=True)
    h = jnp.where(msk, jnp.exp(ca - dmax), 0.0)
    g_ref[0] = g
    h_ref[0] = h
    zero5 = jnp.zeros((vb, PSI_W - 3 * M), jnp.float32)
    psi = jnp.concatenate([g[:, m:m + 1] * xb for m in range(M)] + [zero5], axis=1)
    hx = jnp.concatenate(
        [jnp.broadcast_to(h[:, m:m + 1], (vb, 3)) for m in range(M)] + [zero5], axis=1)
    psi_ref[0] = psi
    hx_ref[0] = hx


def _prep_dec_body(d_ref, u_ref, c_ref, wdf_ref, g_ref, h_ref, hx_ref, psi_ref):
    db = d_ref[0]
    a = jnp.dot(db, u_ref[...], preferred_element_type=jnp.float32)
    vb = a.shape[0]
    msk = _lane_mask((vb, 16), M, 1)
    amax = jnp.max(jnp.where(msk, a, NEG), axis=1, keepdims=True)
    g = jnp.where(msk, jnp.exp(a - amax), 0.0)
    ca = c_ref[...][None, :] - a
    dmax = jnp.max(jnp.where(msk, ca, NEG), axis=1, keepdims=True)
    h = jnp.where(msk, jnp.exp(ca - dmax), 0.0)
    g_ref[0] = g
    h_ref[0] = h
    p2 = jnp.dot(db, wdf_ref[...], preferred_element_type=jnp.float32)
    zero5 = jnp.zeros((vb, PSI_W - 3 * M), jnp.float32)
    gx = jnp.concatenate(
        [jnp.broadcast_to(g[:, m:m + 1], (vb, 3)) for m in range(M)] + [zero5], axis=1)
    hx = jnp.concatenate(
        [jnp.broadcast_to(h[:, m:m + 1], (vb, 3)) for m in range(M)] + [zero5], axis=1)
    psi_ref[0] = gx * p2
    hx_ref[0] = hx


def _post_enc_body(y_ref, hx_ref, wf_ref, b_ref, o_ref):
    t = y_ref[0] * hx_ref[0]
    h1 = jnp.dot(t, wf_ref[...], preferred_element_type=jnp.float32)
    o_ref[0] = jnp.maximum(h1 + b_ref[...][None, :], 0.0)


def _post_dec_body(y_ref, hx_ref, s_ref, b_ref, o_ref):
    t = y_ref[0] * hx_ref[0]
    o = jnp.dot(t, s_ref[...], preferred_element_type=jnp.float32)
    o_ref[0] = jnp.maximum(o + b_ref[...][None, :], 0.0)


def _latent_body(h_ref, w_ref, b_ref, z_ref):
    k = pl.program_id(0)

    @pl.when(k == 0)
    def _():
        z_ref[...] = jnp.zeros_like(z_ref)

    z_ref[...] += jnp.dot(h_ref[...], w_ref[...], preferred_element_type=jnp.float32)

    @pl.when(k == pl.num_programs(0) - 1)
    def _():
        z_ref[...] += b_ref[...]


def _dec_mm_body(z_ref, w_ref, b_ref, d_ref):
    d_ref[...] = jnp.dot(z_ref[...], w_ref[...],
                         preferred_element_type=jnp.float32) + b_ref[...]


def _sc_edge_body(rows_h, cols_h, w_h, g_h, h_h, psi_h, y_h,
                  ysh, rv, cv, wv, cb, rb, rs, gr, hr, pr, msg,
                  sem_r, sem_c, sem_w, sem_g, sem_h, sem_p):
    c = lax.axis_index("c")
    s = lax.axis_index("s")

    zeros16 = jnp.zeros((L,), jnp.float32)

    def zfill(i, _):
        msg[i, pl.ds(0, L)] = zeros16
        msg[i, pl.ds(L, L)] = zeros16
        return 0

    lax.fori_loop(0, CHUNK, zfill, 0)
    for j in range(STRIPE // CHUNK):
        pltpu.sync_copy(msg, ysh.at[pl.ds(s * STRIPE + j * CHUNK, CHUNK)])
    plsc.subcore_barrier()

    n_extra = N_CHUNKS % NS
    n_chunks = jnp.where(s < n_extra, N_CHUNKS // NS + 1, N_CHUNKS // NS)

    def chunk_body(ci, _):
        base = (ci * NS + s) * CHUNK
        dr = pltpu.async_copy(rows_h.at[pl.ds(base, CHUNK)], rv, sem_r)
        dc = pltpu.async_copy(cols_h.at[pl.ds(base, CHUNK)], cv, sem_c)
        dw = pltpu.async_copy(w_h.at[pl.ds(base, CHUNK)], wv.at[pl.ds(0, CHUNK)],
                              sem_w)
        dr.wait()
        dc.wait()
        dw.wait()
        for bl in range(2):
            b = 2 * c + bl
            bv = b * V
            for j in range(CHUNK // L):
                sl = pl.ds(j * L, L)
                cvv = cv[sl]
                rvv = rv[sl]
                cb[sl] = cvv + bv
                rb[sl] = rvv + bv
                rs[sl] = rvv + bl * V
            dg = pltpu.async_copy(g_h.at[cb], gr, sem_g)
            dh = pltpu.async_copy(h_h.at[rb], hr, sem_h)
            dp = pltpu.async_copy(psi_h.at[cb], pr, sem_p)
            dg.wait()
            dh.wait()
            dp.wait()

            iolane = lax.iota(jnp.int32, L)

            def grp_body(grp, _):
                wvec = wv[pl.ds(grp * 8, L)]
                for k in range(8):
                    e = grp * 8 + k
                    t = gr[e] * hr[e]
                    for sh in (1, 2, 4, 8):
                        t = t + t.at[iolane ^ sh].get(mode="promise_in_bounds")
                    zeta = wvec[k] / t
                    msg[e, pl.ds(0, L)] = pr[e, pl.ds(0, L)] * zeta
                    msg[e, pl.ds(L, L)] = pr[e, pl.ds(L, L)] * zeta
                return 0

            lax.fori_loop(0, CHUNK // 8, grp_body, 0)
            pltpu.sync_copy(msg, ysh.at[rs], add=True)
        return 0

    lax.fori_loop(0, n_chunks, chunk_body, 0)
    plsc.subcore_barrier()
    for j in range(STRIPE // CHUNK):
        off = s * STRIPE + j * CHUNK
        pltpu.sync_copy(ysh.at[pl.ds(off, CHUNK)], msg)
        pltpu.sync_copy(msg, y_h.at[pl.ds(c * YROWS + off, CHUNK)])


def _sc_edge(rows, cols, w, g, h, psi):
    mesh = plsc.VectorSubcoreMesh(core_axis_name="c", subcore_axis_name="s")
    fn = pl.kernel(
        _sc_edge_body,
        out_type=jax.ShapeDtypeStruct((NC * YROWS, PSI_W), jnp.float32),
        mesh=mesh,
        compiler_params=pltpu.CompilerParams(use_tc_tiling_on_sc=False),
        scratch_types=[
            pltpu.VMEM_SHARED((YROWS, PSI_W), jnp.float32),
            pltpu.VMEM((CHUNK,), jnp.int32),
            pltpu.VMEM((CHUNK,), jnp.int32),
            pltpu.VMEM((CHUNK + L,), jnp.float32),
            pltpu.VMEM((CHUNK,), jnp.int32),
            pltpu.VMEM((CHUNK,), jnp.int32),
            pltpu.VMEM((CHUNK,), jnp.int32),
            pltpu.VMEM((CHUNK, L), jnp.float32),
            pltpu.VMEM((CHUNK, L), jnp.float32),
            pltpu.VMEM((CHUNK, PSI_W), jnp.float32),
            pltpu.VMEM((CHUNK, PSI_W), jnp.float32),
            pltpu.SemaphoreType.DMA,
            pltpu.SemaphoreType.DMA,
            pltpu.SemaphoreType.DMA,
            pltpu.SemaphoreType.DMA,
            pltpu.SemaphoreType.DMA,
            pltpu.SemaphoreType.DMA,
        ],
    )
    ypad = fn(rows, cols, w, g, h, psi)
    return jnp.concatenate([ypad[:2 * V], ypad[YROWS:YROWS + 2 * V]], axis=0)


VB = 2000
NB = V // VB


def _node_specs(widths):
    return [pl.BlockSpec((1, VB, w), lambda b, i: (b, i, 0)) for w in widths]


def kernel(x, edge_index, edge_weight, W_e, u_e, c_e, b_e, W_lat, b_lat,
           W_dec, b_dec, W_d, u_d, c_d, b_d):
    f32 = jnp.float32
    rows = edge_index[0]
    cols = edge_index[1]

    u_e_pad = jnp.zeros((F_IN, 16), f32).at[:, :M].set(u_e)
    c_e_pad = jnp.zeros((16,), f32).at[:M].set(c_e)
    g1, h1t, hx1, psi1 = pl.pallas_call(
        _prep_enc_body,
        grid=(B, NB),
        in_specs=[
            pl.BlockSpec((1, VB, F_IN), lambda b, i: (b, i, 0)),
            pl.BlockSpec((F_IN, 16), lambda b, i: (0, 0)),
            pl.BlockSpec((16,), lambda b, i: (0,)),
        ],
        out_specs=_node_specs([16, 16, PSI_W, PSI_W]),
        out_shape=[
            jax.ShapeDtypeStruct((B, V, 16), f32),
            jax.ShapeDtypeStruct((B, V, 16), f32),
            jax.ShapeDtypeStruct((B, V, PSI_W), f32),
            jax.ShapeDtypeStruct((B, V, PSI_W), f32),
        ],
    )(x, u_e_pad, c_e_pad)

    y1 = _sc_edge(rows, cols, edge_weight,
                  g1.reshape(B * V, 16), h1t.reshape(B * V, 16),
                  psi1.reshape(B * V, PSI_W))
    y1 = y1.reshape(B, V, PSI_W)

    wf = jnp.zeros((PSI_W, F_MID), f32).at[:3 * M, :].set(W_e.reshape(3 * M, F_MID))
    h1 = pl.pallas_call(
        _post_enc_body,
        grid=(B, NB),
        in_specs=_node_specs([PSI_W, PSI_W]) + [
            pl.BlockSpec((PSI_W, F_MID), lambda b, i: (0, 0)),
            pl.BlockSpec((F_MID,), lambda b, i: (0,)),
        ],
        out_specs=_node_specs([F_MID])[0],
        out_shape=jax.ShapeDtypeStruct((B, V, F_MID), f32),
    )(y1, hx1, wf, b_e)

    h1f = h1.reshape(B, V * F_MID)
    KB = 16000
    NKB = (V * F_MID) // KB
    z = pl.pallas_call(
        _latent_body,
        grid=(NKB,),
        in_specs=[
            pl.BlockSpec((B, KB), lambda k: (0, k)),
            pl.BlockSpec((KB, LATENT), lambda k: (k, 0)),
            pl.BlockSpec((1, LATENT), lambda k: (0, 0)),
        ],
        out_specs=pl.BlockSpec((B, LATENT), lambda k: (0, 0)),
        out_shape=jax.ShapeDtypeStruct((B, LATENT), f32),
    )(h1f, W_lat, b_lat.reshape(1, LATENT))

    NB2 = 10
    NBW = (V * F_MID) // NB2
    d = pl.pallas_call(
        _dec_mm_body,
        grid=(NB2,),
        in_specs=[
            pl.BlockSpec((B, LATENT), lambda n: (0, 0)),
            pl.BlockSpec((LATENT, NBW), lambda n: (0, n)),
            pl.BlockSpec((1, NBW), lambda n: (0, n)),
        ],
        out_specs=pl.BlockSpec((B, NBW), lambda n: (0, n)),
        out_shape=jax.ShapeDtypeStruct((B, V * F_MID), f32),
    )(z, W_dec, b_dec.reshape(1, V * F_MID))
    d = d.reshape(B, V, F_MID)

    u_d_pad = jnp.zeros((F_MID, 16), f32).at[:, :M].set(u_d)
    c_d_pad = jnp.zeros((16,), f32).at[:M].set(c_d)
    wdf = jnp.zeros((F_MID, PSI_W), f32).at[:, :3 * M].set(
        W_d.transpose(1, 0, 2).reshape(F_MID, 3 * M))
    g2, h2t, hx2, psi2 = pl.pallas_call(
        _prep_dec_body,
        grid=(B, NB),
        in_specs=[
            pl.BlockSpec((1, VB, F_MID), lambda b, i: (b, i, 0)),
            pl.BlockSpec((F_MID, 16), lambda b, i: (0, 0)),
            pl.BlockSpec((16,), lambda b, i: (0,)),
            pl.BlockSpec((F_MID, PSI_W), lambda b, i: (0, 0)),
        ],
        out_specs=_node_specs([16, 16, PSI_W, PSI_W]),
        out_shape=[
            jax.ShapeDtypeStruct((B, V, 16), f32),
            jax.ShapeDtypeStruct((B, V, 16), f32),
            jax.ShapeDtypeStruct((B, V, PSI_W), f32),
            jax.ShapeDtypeStruct((B, V, PSI_W), f32),
        ],
    )(d, u_d_pad, c_d_pad, wdf)

    y2 = _sc_edge(rows, cols, edge_weight,
                  g2.reshape(B * V, 16), h2t.reshape(B * V, 16),
                  psi2.reshape(B * V, PSI_W))
    y2 = y2.reshape(B, V, PSI_W)

    smat = jnp.asarray(np.tile(np.eye(F_IN, dtype=np.float32), (M, 1)))
    smat = jnp.zeros((PSI_W, F_IN), f32).at[:3 * M, :].set(smat)
    out = pl.pallas_call(
        _post_dec_body,
        grid=(B, NB),
        in_specs=_node_specs([PSI_W, PSI_W]) + [
            pl.BlockSpec((PSI_W, F_IN), lambda b, i: (0, 0)),
            pl.BlockSpec((F_IN,), lambda b, i: (0,)),
        ],
        out_specs=_node_specs([F_IN])[0],
        out_shape=jax.ShapeDtypeStruct((B, V, F_IN), f32),
    )(y2, hx2, smat, b_d)
    return out


# speedup vs baseline: 13.6546x; 1.1023x over previous
import functools

import jax
import jax.numpy as jnp
import numpy as np
from jax import lax
from jax.experimental import pallas as pl
from jax.experimental.pallas import tpu as pltpu
from jax.experimental.pallas import tpu_sc as plsc

V = 10000
E = 160000
B = 4
M = 9
F_IN = 3
F_MID = 16
LATENT = 64

NC = 2
NS = 16
L = 16

CHUNK = 128
N_CHUNKS = E // CHUNK
STRIPE = 1280
YROWS = NS * STRIPE
PSI_W = 32
NEG = -1e30


def _lane_mask(shape, n, dim):
    return lax.broadcasted_iota(jnp.int32, shape, dim) < n


def _prep_enc_body(x_ref, u_ref, c_ref, g_ref, h_ref, hx_ref, psi_ref):
    xb = x_ref[0]
    a = jnp.dot(xb, u_ref[...], preferred_element_type=jnp.float32)
    vb = a.shape[0]
    msk = _lane_mask((vb, 16), M, 1)
    amax = jnp.max(jnp.where(msk, a, NEG), axis=1, keepdims=True)
    g = jnp.where(msk, jnp.exp(a - amax), 0.0)
    ca = c_ref[...][None, :] - a
    dmax = jnp.max(jnp.where(msk, ca, NEG), axis=1, keepdims=True)
    h = jnp.where(msk, jnp.exp(ca - dmax), 0.0)
    g_ref[0] = g
    h_ref[0] = h
    zero5 = jnp.zeros((vb, PSI_W - 3 * M), jnp.float32)
    psi = jnp.concatenate([g[:, m:m + 1] * xb for m in range(M)] + [zero5], axis=1)
    hx = jnp.concatenate(
        [jnp.broadcast_to(h[:, m:m + 1], (vb, 3)) for m in range(M)] + [zero5], axis=1)
    psi_ref[0] = psi
    hx_ref[0] = hx


def _prep_dec_body(d_ref, u_ref, c_ref, wdf_ref, g_ref, h_ref, hx_ref, psi_ref):
    db = d_ref[0]
    a = jnp.dot(db, u_ref[...], preferred_element_type=jnp.float32)
    vb = a.shape[0]
    msk = _lane_mask((vb, 16), M, 1)
    amax = jnp.max(jnp.where(msk, a, NEG), axis=1, keepdims=True)
    g = jnp.where(msk, jnp.exp(a - amax), 0.0)
    ca = c_ref[...][None, :] - a
    dmax = jnp.max(jnp.where(msk, ca, NEG), axis=1, keepdims=True)
    h = jnp.where(msk, jnp.exp(ca - dmax), 0.0)
    g_ref[0] = g
    h_ref[0] = h
    p2 = jnp.dot(db, wdf_ref[...], preferred_element_type=jnp.float32)
    zero5 = jnp.zeros((vb, PSI_W - 3 * M), jnp.float32)
    gx = jnp.concatenate(
        [jnp.broadcast_to(g[:, m:m + 1], (vb, 3)) for m in range(M)] + [zero5], axis=1)
    hx = jnp.concatenate(
        [jnp.broadcast_to(h[:, m:m + 1], (vb, 3)) for m in range(M)] + [zero5], axis=1)
    psi_ref[0] = gx * p2
    hx_ref[0] = hx


def _post_enc_body(y_ref, hx_ref, wf_ref, b_ref, o_ref):
    t = y_ref[0] * hx_ref[0]
    h1 = jnp.dot(t, wf_ref[...], preferred_element_type=jnp.float32)
    o_ref[0] = jnp.maximum(h1 + b_ref[...][None, :], 0.0)


def _post_dec_body(y_ref, hx_ref, s_ref, b_ref, o_ref):
    t = y_ref[0] * hx_ref[0]
    o = jnp.dot(t, s_ref[...], preferred_element_type=jnp.float32)
    o_ref[0] = jnp.maximum(o + b_ref[...][None, :], 0.0)


def _latent_body(h_ref, w_ref, b_ref, z_ref):
    k = pl.program_id(0)

    @pl.when(k == 0)
    def _():
        z_ref[...] = jnp.zeros_like(z_ref)

    z_ref[...] += jnp.dot(h_ref[...], w_ref[...], preferred_element_type=jnp.float32)

    @pl.when(k == pl.num_programs(0) - 1)
    def _():
        z_ref[...] += b_ref[...]


def _dec_mm_body(z_ref, w_ref, b_ref, d_ref):
    d_ref[...] = jnp.dot(z_ref[...], w_ref[...],
                         preferred_element_type=jnp.float32) + b_ref[...]


def _sc_edge_body(rows_h, cols_h, w_h, g_h, h_h, psi_h, y_h,
                  ysh, rv, cv, wv, wcur,
                  cb0, rb0, rs0, cb1, rb1, rs1,
                  gr0, hr0, pr0, gr1, hr1, pr1, msg,
                  sem_r, sem_c, sem_w,
                  sem_g0, sem_h0, sem_p0, sem_g1, sem_h1, sem_p1):
    c = lax.axis_index("c")
    s = lax.axis_index("s")
    cbs = (cb0, cb1)
    rbs = (rb0, rb1)
    rss = (rs0, rs1)
    grs = (gr0, gr1)
    hrs = (hr0, hr1)
    prs = (pr0, pr1)
    sgs = (sem_g0, sem_g1)
    shs = (sem_h0, sem_h1)
    sps = (sem_p0, sem_p1)

    zeros16 = jnp.zeros((L,), jnp.float32)

    def zfill(i, _):
        msg[i, pl.ds(0, L)] = zeros16
        msg[i, pl.ds(L, L)] = zeros16
        return 0

    lax.fori_loop(0, CHUNK, zfill, 0)
    for j in range(STRIPE // CHUNK):
        pltpu.sync_copy(msg, ysh.at[pl.ds(s * STRIPE + j * CHUNK, CHUNK)])
    plsc.subcore_barrier()

    n_extra = N_CHUNKS % NS
    n_chunks = jnp.where(s < n_extra, N_CHUNKS // NS + 1, N_CHUNKS // NS)
    iolane = lax.iota(jnp.int32, L)

    def issue_idx(ci):
        base = (ci * NS + s) * CHUNK
        pltpu.async_copy(rows_h.at[pl.ds(base, CHUNK)], rv, sem_r)
        pltpu.async_copy(cols_h.at[pl.ds(base, CHUNK)], cv, sem_c)
        pltpu.async_copy(w_h.at[pl.ds(base, CHUNK)], wv, sem_w)

    def wait_idx(ci):
        base = (ci * NS + s) * CHUNK
        pltpu.make_async_copy(rows_h.at[pl.ds(base, CHUNK)], rv, sem_r).wait()
        pltpu.make_async_copy(cols_h.at[pl.ds(base, CHUNK)], cv, sem_c).wait()
        pltpu.make_async_copy(w_h.at[pl.ds(base, CHUNK)], wv, sem_w).wait()

    def adjust_and_gather():
        for bl in range(2):
            b = 2 * c + bl
            bv = b * V
            for j in range(CHUNK // L):
                sl = pl.ds(j * L, L)
                cvv = cv[sl]
                rvv = rv[sl]
                cbs[bl][sl] = cvv + bv
                rbs[bl][sl] = rvv + bv
                rss[bl][sl] = rvv + bl * V
        for j in range(CHUNK // L):
            sl = pl.ds(j * L, L)
            wcur[sl] = wv[sl]
        for bl in range(2):
            pltpu.async_copy(g_h.at[cbs[bl]], grs[bl], sgs[bl])
            pltpu.async_copy(h_h.at[rbs[bl]], hrs[bl], shs[bl])
            pltpu.async_copy(psi_h.at[cbs[bl]], prs[bl], sps[bl])

    issue_idx(0)
    wait_idx(0)
    adjust_and_gather()

    @pl.when(n_chunks > 1)
    def _():
        issue_idx(1)

    def chunk_body(ci, _):
        for bl in range(2):
            pltpu.make_async_copy(g_h.at[cbs[bl]], grs[bl], sgs[bl]).wait()
            pltpu.make_async_copy(h_h.at[rbs[bl]], hrs[bl], shs[bl]).wait()
            pltpu.make_async_copy(psi_h.at[cbs[bl]], prs[bl], sps[bl]).wait()
            gr = grs[bl]
            hr = hrs[bl]
            pr = prs[bl]

            def grp_body(grp, _):
                wvec = wcur[pl.ds(grp * L, L)]
                for k in range(L):
                    e = grp * L + k
                    t = gr[e] * hr[e]
                    for sh in (1, 2, 4, 8):
                        t = t + t.at[iolane ^ sh].get(mode="promise_in_bounds")
                    zeta = wvec[k] / t
                    msg[e, pl.ds(0, L)] = pr[e, pl.ds(0, L)] * zeta
                    msg[e, pl.ds(L, L)] = pr[e, pl.ds(L, L)] * zeta
                return 0

            lax.fori_loop(0, CHUNK // L, grp_body, 0)
            pltpu.sync_copy(msg, ysh.at[rss[bl]], add=True)

        @pl.when(ci + 1 < n_chunks)
        def _():
            wait_idx(ci + 1)
            adjust_and_gather()

        @pl.when(ci + 2 < n_chunks)
        def _():
            issue_idx(ci + 2)

        return 0

    lax.fori_loop(0, n_chunks, chunk_body, 0)
    plsc.subcore_barrier()
    for j in range(STRIPE // CHUNK):
        off = s * STRIPE + j * CHUNK
        pltpu.sync_copy(ysh.at[pl.ds(off, CHUNK)], msg)
        pltpu.sync_copy(msg, y_h.at[pl.ds(c * YROWS + off, CHUNK)])


def _sc_edge(rows, cols, w, g, h, psi):
    mesh = plsc.VectorSubcoreMesh(core_axis_name="c", subcore_axis_name="s")
    fn = pl.kernel(
        _sc_edge_body,
        out_type=jax.ShapeDtypeStruct((NC * YROWS, PSI_W), jnp.float32),
        mesh=mesh,
        compiler_params=pltpu.CompilerParams(use_tc_tiling_on_sc=False),
        scratch_types=[
            pltpu.VMEM_SHARED((YROWS, PSI_W), jnp.float32),
            pltpu.VMEM((CHUNK,), jnp.int32),
            pltpu.VMEM((CHUNK,), jnp.int32),
            pltpu.VMEM((CHUNK,), jnp.float32),
            pltpu.VMEM((CHUNK,), jnp.float32),
            pltpu.VMEM((CHUNK,), jnp.int32),
            pltpu.VMEM((CHUNK,), jnp.int32),
            pltpu.VMEM((CHUNK,), jnp.int32),
            pltpu.VMEM((CHUNK,), jnp.int32),
            pltpu.VMEM((CHUNK,), jnp.int32),
            pltpu.VMEM((CHUNK,), jnp.int32),
            pltpu.VMEM((CHUNK, L), jnp.float32),
            pltpu.VMEM((CHUNK, L), jnp.float32),
            pltpu.VMEM((CHUNK, PSI_W), jnp.float32),
            pltpu.VMEM((CHUNK, L), jnp.float32),
            pltpu.VMEM((CHUNK, L), jnp.float32),
            pltpu.VMEM((CHUNK, PSI_W), jnp.float32),
            pltpu.VMEM((CHUNK, PSI_W), jnp.float32),
            pltpu.SemaphoreType.DMA,
            pltpu.SemaphoreType.DMA,
            pltpu.SemaphoreType.DMA,
            pltpu.SemaphoreType.DMA,
            pltpu.SemaphoreType.DMA,
            pltpu.SemaphoreType.DMA,
            pltpu.SemaphoreType.DMA,
            pltpu.SemaphoreType.DMA,
            pltpu.SemaphoreType.DMA,
        ],
    )
    ypad = fn(rows, cols, w, g, h, psi)
    return jnp.concatenate([ypad[:2 * V], ypad[YROWS:YROWS + 2 * V]], axis=0)


VB = 2000
NB = V // VB


def _node_specs(widths):
    return [pl.BlockSpec((1, VB, w), lambda b, i: (b, i, 0)) for w in widths]


def kernel(x, edge_index, edge_weight, W_e, u_e, c_e, b_e, W_lat, b_lat,
           W_dec, b_dec, W_d, u_d, c_d, b_d):
    f32 = jnp.float32
    rows = edge_index[0]
    cols = edge_index[1]

    u_e_pad = jnp.zeros((F_IN, 16), f32).at[:, :M].set(u_e)
    c_e_pad = jnp.zeros((16,), f32).at[:M].set(c_e)
    g1, h1t, hx1, psi1 = pl.pallas_call(
        _prep_enc_body,
        grid=(B, NB),
        in_specs=[
            pl.BlockSpec((1, VB, F_IN), lambda b, i: (b, i, 0)),
            pl.BlockSpec((F_IN, 16), lambda b, i: (0, 0)),
            pl.BlockSpec((16,), lambda b, i: (0,)),
        ],
        out_specs=_node_specs([16, 16, PSI_W, PSI_W]),
        out_shape=[
            jax.ShapeDtypeStruct((B, V, 16), f32),
            jax.ShapeDtypeStruct((B, V, 16), f32),
            jax.ShapeDtypeStruct((B, V, PSI_W), f32),
            jax.ShapeDtypeStruct((B, V, PSI_W), f32),
        ],
    )(x, u_e_pad, c_e_pad)

    y1 = _sc_edge(rows, cols, edge_weight,
                  g1.reshape(B * V, 16), h1t.reshape(B * V, 16),
                  psi1.reshape(B * V, PSI_W))
    y1 = y1.reshape(B, V, PSI_W)

    wf = jnp.zeros((PSI_W, F_MID), f32).at[:3 * M, :].set(W_e.reshape(3 * M, F_MID))
    h1 = pl.pallas_call(
        _post_enc_body,
        grid=(B, NB),
        in_specs=_node_specs([PSI_W, PSI_W]) + [
            pl.BlockSpec((PSI_W, F_MID), lambda b, i: (0, 0)),
            pl.BlockSpec((F_MID,), lambda b, i: (0,)),
        ],
        out_specs=_node_specs([F_MID])[0],
        out_shape=jax.ShapeDtypeStruct((B, V, F_MID), f32),
    )(y1, hx1, wf, b_e)

    h1f = h1.reshape(B, V * F_MID)
    KB = 16000
    NKB = (V * F_MID) // KB
    z = pl.pallas_call(
        _latent_body,
        grid=(NKB,),
        in_specs=[
            pl.BlockSpec((B, KB), lambda k: (0, k)),
            pl.BlockSpec((KB, LATENT), lambda k: (k, 0)),
            pl.BlockSpec((1, LATENT), lambda k: (0, 0)),
        ],
        out_specs=pl.BlockSpec((B, LATENT), lambda k: (0, 0)),
        out_shape=jax.ShapeDtypeStruct((B, LATENT), f32),
    )(h1f, W_lat, b_lat.reshape(1, LATENT))

    NB2 = 10
    NBW = (V * F_MID) // NB2
    d = pl.pallas_call(
        _dec_mm_body,
        grid=(NB2,),
        in_specs=[
            pl.BlockSpec((B, LATENT), lambda n: (0, 0)),
            pl.BlockSpec((LATENT, NBW), lambda n: (0, n)),
            pl.BlockSpec((1, NBW), lambda n: (0, n)),
        ],
        out_specs=pl.BlockSpec((B, NBW), lambda n: (0, n)),
        out_shape=jax.ShapeDtypeStruct((B, V * F_MID), f32),
    )(z, W_dec, b_dec.reshape(1, V * F_MID))
    d = d.reshape(B, V, F_MID)

    u_d_pad = jnp.zeros((F_MID, 16), f32).at[:, :M].set(u_d)
    c_d_pad = jnp.zeros((16,), f32).at[:M].set(c_d)
    wdf = jnp.zeros((F_MID, PSI_W), f32).at[:, :3 * M].set(
        W_d.transpose(1, 0, 2).reshape(F_MID, 3 * M))
    g2, h2t, hx2, psi2 = pl.pallas_call(
        _prep_dec_body,
        grid=(B, NB),
        in_specs=[
            pl.BlockSpec((1, VB, F_MID), lambda b, i: (b, i, 0)),
            pl.BlockSpec((F_MID, 16), lambda b, i: (0, 0)),
            pl.BlockSpec((16,), lambda b, i: (0,)),
            pl.BlockSpec((F_MID, PSI_W), lambda b, i: (0, 0)),
        ],
        out_specs=_node_specs([16, 16, PSI_W, PSI_W]),
        out_shape=[
            jax.ShapeDtypeStruct((B, V, 16), f32),
            jax.ShapeDtypeStruct((B, V, 16), f32),
            jax.ShapeDtypeStruct((B, V, PSI_W), f32),
            jax.ShapeDtypeStruct((B, V, PSI_W), f32),
        ],
    )(d, u_d_pad, c_d_pad, wdf)

    y2 = _sc_edge(rows, cols, edge_weight,
                  g2.reshape(B * V, 16), h2t.reshape(B * V, 16),
                  psi2.reshape(B * V, PSI_W))
    y2 = y2.reshape(B, V, PSI_W)

    smat = jnp.asarray(np.tile(np.eye(F_IN, dtype=np.float32), (M, 1)))
    smat = jnp.zeros((PSI_W, F_IN), f32).at[:3 * M, :].set(smat)
    out = pl.pallas_call(
        _post_dec_body,
        grid=(B, NB),
        in_specs=_node_specs([PSI_W, PSI_W]) + [
            pl.BlockSpec((PSI_W, F_IN), lambda b, i: (0, 0)),
            pl.BlockSpec((F_IN,), lambda b, i: (0,)),
        ],
        out_specs=_node_specs([F_IN])[0],
        out_shape=jax.ShapeDtypeStruct((B, V, F_IN), f32),
    )(y2, hx2, smat, b_d)
    return out
